# baseline (speedup 1.0000x reference)
import jax
import jax.numpy as jnp
from jax.experimental import pallas as pl


_ROWS, _COLS = 16384, 128
_BLOCK_ROWS = 2048
_GRID = _ROWS // _BLOCK_ROWS


def _copy_set_kernel(x_ref, o_ref):
    blk = x_ref[...]

    @pl.when(pl.program_id(0) == 0)
    def _():
        rows = jax.lax.broadcasted_iota(jnp.int32, (_BLOCK_ROWS, _COLS), 0)
        cols = jax.lax.broadcasted_iota(jnp.int32, (_BLOCK_ROWS, _COLS), 1)
        hit = (rows == 0) & (cols == 0)
        o_ref[...] = jnp.where(hit, jnp.float32(3.0), blk)

    @pl.when(pl.program_id(0) != 0)
    def _():
        o_ref[...] = blk


def kernel(x):
    return pl.pallas_call(
        _copy_set_kernel,
        grid=(_GRID,),
        in_specs=[pl.BlockSpec((_BLOCK_ROWS, _COLS), lambda i: (i, 0))],
        out_specs=pl.BlockSpec((_BLOCK_ROWS, _COLS), lambda i: (i, 0)),
        out_shape=jax.ShapeDtypeStruct((_ROWS, _COLS), x.dtype),
    )(x)
